# Initial kernel scaffold; baseline (speedup 1.0000x reference)
import jax
import jax.numpy as jnp
from jax.experimental import pallas as pl


def kernel(target, pos_context, neg_context, in_W, out_W):
    raise NotImplementedError("write your pallas kernel here")


# speedup vs baseline: 1.7566x; 1.7566x over previous
import dataclasses

import jax
import jax.numpy as jnp
from jax import lax
from jax.experimental import pallas as pl
from jax.experimental.pallas import tpu as pltpu
from jax.experimental.pallas import tpu_sc as plsc

NC, NS, L = 2, 16, 16
NW = NC * NS
B = 4096
D = 64
N_POS = 20
N_CTX = 70
N_PAD = 72
B_PER_W = B // NW
CHUNK = 8
N_CHUNKS = B_PER_W // CHUNK


def _sc_body(tgt_hbm, ctx_hbm, inW_hbm, outW_hbm, scores_hbm,
             idx_t_v, v_rows, idx_c, u_buf, score_v, sem, gsem):
    wid = lax.axis_index("s") * NC + lax.axis_index("c")
    base = wid * B_PER_W

    pltpu.sync_copy(tgt_hbm.at[pl.ds(base, B_PER_W)], idx_t_v)
    pltpu.async_copy(inW_hbm.at[idx_t_v], v_rows, gsem).wait()

    lane = lax.iota(jnp.int32, L)
    m_last = lane == (L - 1)

    @pl.loop(0, N_CHUNKS)
    def _(cb):
        row0 = base + cb * CHUNK
        pltpu.sync_copy(ctx_hbm.at[pl.ds(row0 * N_PAD, CHUNK * N_PAD)], idx_c)
        copies = []
        for j in range(CHUNK):
            cp = pltpu.make_async_copy(
                outW_hbm.at[idx_c.at[pl.ds(j * N_PAD, N_PAD)]],
                u_buf.at[pl.ds(j * N_PAD, N_PAD)],
                sem)
            cp.start()
            copies.append(cp)
        for cp in copies:
            cp.wait()

        for j in range(CHUNK):
            vrow = v_rows.at[cb * CHUNK + j]
            v0 = vrow[pl.ds(0, L)]
            v1 = vrow[pl.ds(16, L)]
            v2 = vrow[pl.ds(32, L)]
            v3 = vrow[pl.ds(48, L)]
            j_splat = jnp.full((L,), j, jnp.int32)

            @pl.loop(0, N_CTX)
            def _(n):
                urow = u_buf.at[j * N_PAD + n]
                acc = ((urow[pl.ds(0, L)] * v0 + urow[pl.ds(16, L)] * v1)
                       + (urow[pl.ds(32, L)] * v2 + urow[pl.ds(48, L)] * v3))
                tot = plsc.cumsum(acc)
                plsc.store_scatter(score_v,
                                   [j_splat, jnp.full((L,), n, jnp.int32)],
                                   tot, mask=m_last)

        pltpu.sync_copy(score_v, scores_hbm.at[pl.ds(row0, CHUNK)])


def kernel(target, pos_context, neg_context, in_W, out_W):
    ctx = jnp.concatenate(
        [pos_context, neg_context, pos_context[:, : N_PAD - N_CTX]], axis=1)
    ctx_flat = ctx.astype(jnp.int32).reshape(-1)
    tgt = target.astype(jnp.int32)

    mesh = plsc.VectorSubcoreMesh(core_axis_name="c", subcore_axis_name="s",
                                  num_cores=NC, num_subcores=NS)
    cp = pltpu.CompilerParams()
    if "needs_layout_passes" in pltpu.CompilerParams.__dataclass_fields__:
        cp = dataclasses.replace(cp, needs_layout_passes=False)
    if "use_tc_tiling_on_sc" in pltpu.CompilerParams.__dataclass_fields__:
        cp = dataclasses.replace(cp, use_tc_tiling_on_sc=False)
    scores = pl.kernel(
        _sc_body,
        out_type=jax.ShapeDtypeStruct((B, N_PAD), jnp.float32),
        mesh=mesh,
        compiler_params=cp,
        scratch_types=[
            pltpu.VMEM((B_PER_W,), jnp.int32),
            pltpu.VMEM((B_PER_W, D), jnp.float32),
            pltpu.VMEM((CHUNK * N_PAD,), jnp.int32),
            pltpu.VMEM((CHUNK * N_PAD, D), jnp.float32),
            pltpu.VMEM((CHUNK, N_PAD), jnp.float32),
            pltpu.SemaphoreType.DMA,
            pltpu.SemaphoreType.DMA,
        ],
    )(tgt, ctx_flat, in_W, out_W)

    return scores[:, :N_POS], scores[:, N_POS:N_CTX]
